# baseline (speedup 1.0000x reference)
import functools

import jax
import jax.numpy as jnp
from jax import lax
from jax.experimental import pallas as pl
from jax.experimental.pallas import tpu as pltpu
from jax.experimental.pallas import tpu_sc as plsc

NUM_CORES = 2
NUM_SUBCORES = 16
NUM_WORKERS = NUM_CORES * NUM_SUBCORES
L = 16

CH = 256
NCH = 123

NBUF = 4
SUBW = (96, 104)


@functools.lru_cache(maxsize=None)
def _make_pairize(V, D):
    V2 = V // 2
    per_w = V // NUM_WORKERS
    mesh = plsc.VectorSubcoreMesh(core_axis_name="c", subcore_axis_name="s")

    @functools.partial(
        pl.kernel,
        mesh=mesh,
        out_type=jax.ShapeDtypeStruct((V2, 2 * D), jnp.float32),
        compiler_params=pltpu.CompilerParams(
            use_tc_tiling_on_sc=True, needs_layout_passes=False
        ),
        scratch_types=[
            pltpu.VMEM((2, CH, D), jnp.float32),
            pltpu.VMEM((2, CH // 2, 2 * D), jnp.float32),
            pltpu.SemaphoreType.DMA((2,)),
            pltpu.SemaphoreType.DMA((2,)),
        ],
    )
    def pairize_kernel(table_hbm, out_hbm, buf_a, buf_b, rsem, wsem):
        wid = lax.axis_index("s") * NUM_CORES + lax.axis_index("c")
        start = (wid * per_w) // L * L
        hi = V - CH

        def coff(k):
            return pl.multiple_of(jnp.minimum(start + k * CH, hi), L)

        for p in range(2):
            pltpu.async_copy(
                table_hbm.at[pl.ds(coff(p), CH)], buf_a.at[p], rsem.at[p]
            )

        def shuffle(p):
            def grp(g, c):
                r0 = g * L
                for l in range(L):
                    r = r0 + l
                    for j in range(D // L):
                        buf_b[
                            p, (r0 // 2) + (l // 2), pl.ds((l % 2) * D + j * L, L)
                        ] = buf_a[p, r, pl.ds(j * L, L)]
                return c

            lax.fori_loop(0, CH // L, grp, 0)

        def rounds(m, carry):
            for p in range(2):
                k = m * 2 + p
                pltpu.make_async_copy(
                    table_hbm.at[pl.ds(0, CH)], buf_a.at[p], rsem.at[p]
                ).wait()

                @pl.when(m > 0)
                def _():
                    pltpu.make_async_copy(
                        buf_b.at[p], out_hbm.at[pl.ds(0, CH // 2)], wsem.at[p]
                    ).wait()

                shuffle(p)
                pltpu.async_copy(
                    buf_b.at[p], out_hbm.at[pl.ds(pl.multiple_of(coff(k) // 2, 8), CH // 2)], wsem.at[p]
                )
                pltpu.async_copy(
                    table_hbm.at[pl.ds(coff(k + 2), CH)], buf_a.at[p], rsem.at[p]
                )
            return carry

        lax.fori_loop(0, (NCH - 1) // 2, rounds, 0)
        k = NCH - 1
        pltpu.make_async_copy(
            table_hbm.at[pl.ds(0, CH)], buf_a.at[0], rsem.at[0]
        ).wait()
        pltpu.make_async_copy(
            buf_b.at[0], out_hbm.at[pl.ds(0, CH // 2)], wsem.at[0]
        ).wait()
        shuffle(0)
        pltpu.async_copy(
            buf_b.at[0], out_hbm.at[pl.ds(pl.multiple_of(coff(k) // 2, 8), CH // 2)], wsem.at[0]
        )
        pltpu.make_async_copy(
            table_hbm.at[pl.ds(0, CH)], buf_a.at[1], rsem.at[1]
        ).wait()
        for p in range(2):
            pltpu.make_async_copy(
                buf_b.at[p], out_hbm.at[pl.ds(0, CH // 2)], wsem.at[p]
            ).wait()

    return pairize_kernel


@functools.lru_cache(maxsize=None)
def _make_gather(V2, D, B, T):
    rows_per_w = B // NUM_WORKERS
    assert B == rows_per_w * NUM_WORKERS and rows_per_w % 2 == 0
    n_outer = rows_per_w // 2
    wmax = max(SUBW)
    mesh = plsc.VectorSubcoreMesh(core_axis_name="c", subcore_axis_name="s")

    @functools.partial(
        pl.kernel,
        mesh=mesh,
        out_type=jax.ShapeDtypeStruct((B, T, D), jnp.float32),
        compiler_params=pltpu.CompilerParams(
            use_tc_tiling_on_sc=True, needs_layout_passes=False
        ),
        scratch_types=[
            pltpu.VMEM((2, T), jnp.int32),
            pltpu.VMEM((NBUF, wmax), jnp.int32),
            pltpu.VMEM((NBUF, wmax), jnp.int32),
            pltpu.VMEM((NBUF, wmax, 2 * D), jnp.float32),
            pltpu.VMEM((NBUF, wmax, D), jnp.float32),
            pltpu.SemaphoreType.DMA((2,)),
            pltpu.SemaphoreType.DMA((NBUF,)),
            pltpu.SemaphoreType.DMA((NBUF,)),
        ],
    )
    def gather_kernel(
        idx_hbm, table_hbm, out_hbm, idx_v, hoff_v, pair_v, pairs_v, rows_v,
        isem, gsem, osem
    ):
        wid = lax.axis_index("s") * NUM_CORES + lax.axis_index("c")
        base = wid * rows_per_w
        iota = lax.iota(jnp.int32, L)

        def sub(b):
            return SUBW[b % 2], (0 if b % 2 == 0 else SUBW[0])

        for q in range(2):
            pltpu.async_copy(idx_hbm.at[base + q], idx_v.at[q], isem.at[q])

        def compact16(b, k0, lo, hi):
            hv = hoff_v[b, pl.ds(k0, L)]
            for l in range(lo, hi):
                hb = hv[jnp.full((L,), l, jnp.int32)]
                k = k0 + l
                src = pairs_v.at[b, k]
                for j in range(D // L):
                    rows_v[b, k, pl.ds(j * L, L)] = plsc.load_gather(
                        src, [hb + (j * L + iota)]
                    )

        def outer(go, carry):
            r0 = base + go * 2
            for q in range(2):
                pltpu.make_async_copy(
                    idx_hbm.at[base], idx_v.at[q], isem.at[q]
                ).wait()
                for half in range(2):
                    b = 2 * q + half
                    w, t0 = sub(b)

                    @pl.when(go > 0)
                    def _():
                        pltpu.make_async_copy(
                            rows_v.at[b, pl.ds(0, w)],
                            out_hbm.at[base, pl.ds(t0, w)],
                            osem.at[b],
                        ).wait()

                    starts = [j * L for j in range(w // L)]
                    if w % L:
                        starts.append(w - L)
                    for o in starts:
                        v = idx_v[q, pl.ds(t0 + o, L)]
                        pair_v[b, pl.ds(o, L)] = lax.shift_right_logical(v, 1)
                        hoff_v[b, pl.ds(o, L)] = (v & 1) * D
                    pltpu.async_copy(
                        table_hbm.at[pair_v.at[b, pl.ds(0, w)]],
                        pairs_v.at[b, pl.ds(0, w)],
                        gsem.at[b],
                    )
                nxt = jnp.minimum(r0 + 2, base + rows_per_w - 2) + q
                pltpu.async_copy(idx_hbm.at[nxt], idx_v.at[q], isem.at[q])
            for b in range(NBUF):
                w, t0 = sub(b)
                row = r0 + b // 2
                pltpu.make_async_copy(
                    table_hbm.at[pl.ds(0, w)],
                    pairs_v.at[b, pl.ds(0, w)],
                    gsem.at[b],
                ).wait()

                def grp(g, c):
                    compact16(b, g * L, 0, L)
                    return c

                lax.fori_loop(0, w // L, grp, 0)
                if w % L:
                    compact16(b, w - L, L - (w % L), L)

                pltpu.async_copy(
                    rows_v.at[b, pl.ds(0, w)],
                    out_hbm.at[row, pl.ds(t0, w)],
                    osem.at[b],
                )
            return carry

        lax.fori_loop(0, n_outer, outer, 0)
        for b in range(NBUF):
            w, t0 = sub(b)
            pltpu.make_async_copy(
                rows_v.at[b, pl.ds(0, w)],
                out_hbm.at[base, pl.ds(t0, w)],
                osem.at[b],
            ).wait()
        for q in range(2):
            pltpu.make_async_copy(idx_hbm.at[base], idx_v.at[q], isem.at[q]).wait()

    return gather_kernel


def kernel(input, table):
    B, T = input.shape
    V, D = table.shape
    idx = input.astype(jnp.int32)
    table2 = _make_pairize(V, D)(table)
    return _make_gather(V // 2, D, B, T)(idx, table2)


# speedup vs baseline: 1.2821x; 1.2821x over previous
import functools

import jax
import jax.numpy as jnp
from jax import lax
from jax.experimental import pallas as pl
from jax.experimental.pallas import tpu as pltpu
from jax.experimental.pallas import tpu_sc as plsc

NUM_CORES = 2
NUM_SUBCORES = 16
NUM_WORKERS = NUM_CORES * NUM_SUBCORES
NBUF = 4
SPLIT = (128, 72)


@functools.lru_cache(maxsize=None)
def _make_gather(V, D, B, T):
    assert B % (NUM_WORKERS * NBUF) == 0
    rows_per_w = B // NUM_WORKERS
    n_outer = rows_per_w // NBUF
    mesh = plsc.VectorSubcoreMesh(core_axis_name="c", subcore_axis_name="s")

    @functools.partial(
        pl.kernel,
        mesh=mesh,
        out_type=jax.ShapeDtypeStruct((B, T, D), jnp.float32),
        compiler_params=pltpu.CompilerParams(use_tc_tiling_on_sc=False),
        scratch_types=[
            pltpu.VMEM((NBUF, T), jnp.int32),
            pltpu.VMEM((NBUF, T, D), jnp.float32),
            pltpu.SemaphoreType.DMA((NBUF,)),
            pltpu.SemaphoreType.DMA((NBUF,)),
            pltpu.SemaphoreType.DMA((NBUF,)),
        ],
    )
    def gather_kernel(idx_hbm, table_hbm, out_hbm, idx_v, rows_v, isem, gsem, osem):
        wid = lax.axis_index("s") * NUM_CORES + lax.axis_index("c")
        base = wid * rows_per_w
        last = base + rows_per_w - NBUF

        for b in range(NBUF):
            pltpu.async_copy(idx_hbm.at[base + b], idx_v.at[b], isem.at[b])

        def outer(go, carry):
            r0 = base + go * NBUF
            for b in range(NBUF):

                @pl.when(go > 0)
                def _():
                    pltpu.make_async_copy(
                        rows_v.at[b], out_hbm.at[base], osem.at[b]
                    ).wait()

                pltpu.make_async_copy(idx_hbm.at[base], idx_v.at[b], isem.at[b]).wait()
                o = 0
                for w in SPLIT:
                    pltpu.async_copy(
                        table_hbm.at[idx_v.at[b, pl.ds(o, w)]],
                        rows_v.at[b, pl.ds(o, w)],
                        gsem.at[b],
                    )
                    o += w
            for b in range(NBUF):
                o = 0
                for w in SPLIT:
                    pltpu.make_async_copy(
                        table_hbm.at[pl.ds(0, w)],
                        rows_v.at[b, pl.ds(o, w)],
                        gsem.at[b],
                    ).wait()
                    o += w
                pltpu.async_copy(rows_v.at[b], out_hbm.at[r0 + b], osem.at[b])
                nxt = jnp.minimum(r0 + NBUF, last) + b
                pltpu.async_copy(idx_hbm.at[nxt], idx_v.at[b], isem.at[b])
            return carry

        lax.fori_loop(0, n_outer, outer, 0)
        for b in range(NBUF):
            pltpu.make_async_copy(rows_v.at[b], out_hbm.at[base], osem.at[b]).wait()
            pltpu.make_async_copy(idx_hbm.at[base], idx_v.at[b], isem.at[b]).wait()

    return gather_kernel


def kernel(input, table):
    B, T = input.shape
    D = table.shape[1]
    idx = input.astype(jnp.int32)
    return _make_gather(table.shape[0], D, B, T)(idx, table)
